# Initial kernel scaffold; baseline (speedup 1.0000x reference)
import jax
import jax.numpy as jnp
from jax.experimental import pallas as pl


def kernel(var_name_log_probs, variable_tgt_name_id, variable_tgt_name_weight, restoration_indices, restoration_mask):
    raise NotImplementedError("write your pallas kernel here")


# speedup vs baseline: 1.3005x; 1.3005x over previous
import functools

import jax
import jax.numpy as jnp
from jax import lax
from jax.experimental import pallas as pl
from jax.experimental.pallas import tpu as pltpu
from jax.experimental.pallas import tpu_sc as plsc

TOTAL = 16384
VOCAB = 4096
NAST = 16
MAXV = 2048
NSUB = 16
CHUNK = TOTAL // NSUB
L = 16
NIDX = 128
NCH = CHUNK // NIDX


def _body(tbl, ids, wts, ridx, rmask, out_ast, out_ppl, out_m,
          ids_v, w_v, flat_v, ll_v, wll_v, macc_v, ridx_v, rmask_v,
          wll_full, metrics_l, stage_v,
          wll_sh,
          sem_in, sem_g, sem_r):
    sid = lax.axis_index("s")
    base = sid * CHUNK

    cp_ids = pltpu.async_copy(ids.at[pl.ds(base, CHUNK)], ids_v, sem_in)
    cp_w = pltpu.async_copy(wts.at[pl.ds(base, CHUNK)], w_v, sem_in)
    cp_ri = pltpu.async_copy(ridx.at[sid], ridx_v, sem_r)
    cp_rm = pltpu.async_copy(rmask.at[sid], rmask_v, sem_r)
    cp_ids.wait()
    cp_w.wait()

    iota = lax.iota(jnp.int32, L)
    for j in range(CHUNK // L):
        v = ids_v[pl.ds(j * L, L)]
        rows = (base + j * L) + iota
        flat_v[j // (NIDX // L), pl.ds((j % (NIDX // L)) * L, L)] = rows * VOCAB + v

    gcp = [pltpu.async_copy(tbl.at[flat_v.at[c]],
                            ll_v.at[pl.ds(c * NIDX, NIDX)], sem_g)
           for c in range(NCH)]
    for cp in gcp:
        cp.wait()

    zero = jnp.zeros((L,), jnp.float32)
    one = jnp.ones((L,), jnp.float32)
    sr = zero
    nr = zero
    su = zero
    nu = zero
    for j in range(CHUNK // L):
        ll = ll_v[pl.ds(j * L, L)]
        w = w_v[pl.ds(j * L, L)]
        rm = jnp.where(w == 1.0, one, zero)
        lr = ll * rm
        sr = sr + lr
        nr = nr + rm
        su = su + (ll - lr)
        nu = nu + (one - rm)
        wll_v[pl.ds(j * L, L)] = ll * w
    macc_v[0, :] = sr
    macc_v[1, :] = nr
    macc_v[2, :] = su
    macc_v[3, :] = nu
    pltpu.sync_copy(wll_v, wll_sh.at[pl.ds(base, CHUNK)])
    pltpu.sync_copy(macc_v, out_m.at[sid])
    plsc.subcore_barrier()

    @pl.when(sid == 0)
    def _ppl():
        pltpu.sync_copy(out_m, metrics_l)
        sr_t = zero
        nr_t = zero
        su_t = zero
        nu_t = zero
        for t in range(NSUB):
            sr_t = sr_t + metrics_l[t, 0, :]
            nr_t = nr_t + metrics_l[t, 1, :]
            su_t = su_t + metrics_l[t, 2, :]
            nu_t = nu_t + metrics_l[t, 3, :]
        ssr = jnp.full((L,), jnp.sum(sr_t))
        snr = jnp.full((L,), jnp.sum(nr_t))
        ssu = jnp.full((L,), jnp.sum(su_t))
        snu = jnp.full((L,), jnp.sum(nu_t))
        rv = jnp.exp(-(ssr / snr))
        uv = jnp.exp(-(ssu / snu))
        stage_v[...] = jnp.where(iota == 0, rv, jnp.where(iota == 1, uv, zero))
        pltpu.sync_copy(stage_v, out_ppl)

    pltpu.sync_copy(wll_sh, wll_full)
    cp_ri.wait()
    cp_rm.wait()
    acc = zero
    for k in range(MAXV // L):
        idx = ridx_v[pl.ds(k * L, L)]
        vals = plsc.load_gather(wll_full, [idx])
        m = rmask_v[pl.ds(k * L, L)]
        acc = acc + vals * m
    stage_v[...] = jnp.full((L,), jnp.sum(acc))
    pltpu.sync_copy(stage_v, out_ast.at[sid])


_sc_call = functools.partial(
    pl.kernel,
    out_type=[
        jax.ShapeDtypeStruct((NAST, L), jnp.float32),
        jax.ShapeDtypeStruct((L,), jnp.float32),
        jax.ShapeDtypeStruct((NSUB, 4, L), jnp.float32),
    ],
    mesh=plsc.VectorSubcoreMesh(core_axis_name="c", subcore_axis_name="s",
                                num_cores=1),
    compiler_params=pltpu.CompilerParams(needs_layout_passes=False),
    scratch_types=[
        pltpu.VMEM((CHUNK,), jnp.int32),
        pltpu.VMEM((CHUNK,), jnp.float32),
        pltpu.VMEM((NCH, NIDX), jnp.int32),
        pltpu.VMEM((CHUNK,), jnp.float32),
        pltpu.VMEM((CHUNK,), jnp.float32),
        pltpu.VMEM((4, L), jnp.float32),
        pltpu.VMEM((MAXV,), jnp.int32),
        pltpu.VMEM((MAXV,), jnp.float32),
        pltpu.VMEM((TOTAL,), jnp.float32),
        pltpu.VMEM((NSUB, 4, L), jnp.float32),
        pltpu.VMEM((L,), jnp.float32),
        pltpu.VMEM_SHARED((TOTAL,), jnp.float32),
        pltpu.SemaphoreType.DMA,
        pltpu.SemaphoreType.DMA,
        pltpu.SemaphoreType.DMA,
    ],
)(_body)


def kernel(var_name_log_probs, variable_tgt_name_id, variable_tgt_name_weight,
           restoration_indices, restoration_mask):
    flat_tbl = var_name_log_probs.reshape((TOTAL * VOCAB,))
    out_ast, out_ppl, _ = _sc_call(flat_tbl, variable_tgt_name_id,
                                   variable_tgt_name_weight,
                                   restoration_indices, restoration_mask)
    return (out_ast[:, 0], out_ppl[0], out_ppl[1])
